# Initial kernel scaffold; baseline (speedup 1.0000x reference)
import jax
import jax.numpy as jnp
from jax.experimental import pallas as pl


def kernel(x, Wr, Wg, Wu, Wd):
    raise NotImplementedError("write your pallas kernel here")


# speedup vs baseline: 1.1352x; 1.1352x over previous
import functools

import jax
import jax.numpy as jnp
from jax import lax
from jax.experimental import pallas as pl
from jax.experimental.pallas import tpu as pltpu
from jax.experimental.pallas import tpu_sc as plsc

T, D, FF, E, K = 4096, 1024, 4096, 8, 2
CAP = int(T * K / E * 1.25)
ECAP = E * CAP
N = T * K

NC, NS, L = 2, 16, 16
NW = NC * NS

@functools.cache
def _mesh():
    return plsc.VectorSubcoreMesh(core_axis_name="c", subcore_axis_name="s",
                                  num_cores=NC, num_subcores=NS)


_BT = 512


def _router_body(x_ref, wr_ref, eidx_ref, ew_ref):
    logits = jnp.dot(x_ref[...].astype(jnp.bfloat16),
                     wr_ref[...].astype(jnp.bfloat16),
                     preferred_element_type=jnp.float32)
    col = lax.broadcasted_iota(jnp.int32, (_BT, 128), 1)
    neg = jnp.float32(-jnp.inf)
    lg = jnp.where(col < E, logits, neg)
    m1 = jnp.max(lg, axis=1, keepdims=True)
    i1 = jnp.min(jnp.where(lg == m1, col, 128), axis=1, keepdims=True)
    lg2 = jnp.where(col == i1, neg, lg)
    m2 = jnp.max(lg2, axis=1, keepdims=True)
    i2 = jnp.min(jnp.where(lg2 == m2, col, 128), axis=1, keepdims=True)
    w1 = 1.0 / (1.0 + jnp.exp(m2 - m1))
    eidx_ref[...] = jnp.concatenate([i1, i2], axis=1)
    ew_ref[...] = jnp.concatenate([w1, 1.0 - w1], axis=1)


def _router(x, wr_padded):
    return pl.pallas_call(
        _router_body,
        grid=(T // _BT,),
        in_specs=[
            pl.BlockSpec((_BT, D), lambda i: (i, 0)),
            pl.BlockSpec((D, 128), lambda i: (0, 0)),
        ],
        out_specs=[
            pl.BlockSpec((_BT, 2), lambda i: (i, 0)),
            pl.BlockSpec((_BT, 2), lambda i: (i, 0)),
        ],
        out_shape=[
            jax.ShapeDtypeStruct((T, 2), jnp.int32),
            jax.ShapeDtypeStruct((T, 2), jnp.float32),
        ],
    )(x, wr_padded)


_CH = N // NS
_IVW = ECAP // NS
_TW = T // NS


@functools.cache
def _dispatch_kernel():
  return functools.partial(
    pl.kernel,
    out_type=[
        jax.ShapeDtypeStruct((ECAP,), jnp.int32),
        jax.ShapeDtypeStruct((T,), jnp.int32),
        jax.ShapeDtypeStruct((T,), jnp.int32),
        jax.ShapeDtypeStruct((T,), jnp.float32),
        jax.ShapeDtypeStruct((T,), jnp.float32),
    ],
    mesh=_mesh(),
    compiler_params=pltpu.CompilerParams(needs_layout_passes=False),
    scratch_types=[
        pltpu.VMEM((_CH,), jnp.int32),
        pltpu.VMEM((_CH,), jnp.float32),
        pltpu.VMEM((L,), jnp.int32),
        pltpu.VMEM((NS * L,), jnp.int32),
        pltpu.VMEM((_CH,), jnp.int32),
        pltpu.VMEM((_CH,), jnp.int32),
        pltpu.VMEM((_CH,), jnp.float32),
        pltpu.VMEM((_IVW,), jnp.int32),
        pltpu.VMEM((_TW,), jnp.int32),
        pltpu.VMEM((_TW,), jnp.float32),
        pltpu.VMEM_SHARED((NS * L,), jnp.int32),
        pltpu.VMEM_SHARED((ECAP,), jnp.int32),
    ],
  )(_dispatch_body)


def _dispatch_body(e_hbm, w_hbm, inv_hbm, d0_hbm, d1_hbm, cw0_hbm, cw1_hbm,
              ids_v, w_v, mycnt_v, allcnt_v, dest_v, val_v, cw_v, inv_v,
              dk_v, cwk_v, counts_sh, inv_sh):
    cid = lax.axis_index("c")
    sid = lax.axis_index("s")
    lane = lax.iota(jnp.int32, L)
    one = jnp.full((L,), 1, jnp.int32)
    zero = jnp.zeros((L,), jnp.int32)

    def splat(s):
        return jnp.broadcast_to(s, (L,))

    @pl.when(cid == 0)
    def _():
        base_i = sid * _CH
        pltpu.sync_copy(e_hbm.at[pl.ds(base_i, _CH)], ids_v)
        pltpu.sync_copy(w_hbm.at[pl.ds(base_i, _CH)], w_v)

        def cnt_body(i, accs):
            v = ids_v[pl.ds(i * L, L)]
            return tuple(accs[e] + jnp.where(v == e, one, zero)
                         for e in range(E))

        accs = lax.fori_loop(0, _CH // L, cnt_body,
                             tuple(jnp.zeros((L,), jnp.int32) for _ in range(E)))
        cnt = jnp.zeros((L,), jnp.int32)
        for e in range(E):
            cnt = jnp.where(lane == e, splat(jnp.sum(accs[e])), cnt)
        mycnt_v[...] = cnt
        pltpu.sync_copy(mycnt_v, counts_sh.at[pl.ds(sid * L, L)])

        def zero_body(i, _):
            inv_v[pl.ds(i * L, L)] = jnp.zeros((L,), jnp.int32)
            return 0

        lax.fori_loop(0, _IVW // L, zero_body, 0)
        pltpu.sync_copy(inv_v, inv_sh.at[pl.ds(sid * _IVW, _IVW)])
        plsc.subcore_barrier()

        pltpu.sync_copy(counts_sh, allcnt_v)
        offv = zero
        for w in range(NS):
            offv = offv + jnp.where(jnp.broadcast_to(sid > w, (L,)),
                                    allcnt_v[pl.ds(w * L, L)], zero)
        bases = tuple(jnp.sum(jnp.where(lane == e, offv, zero))
                      for e in range(E))

        def scan_body(i, bases):
            v = ids_v[pl.ds(i * L, L)]
            gi = splat(base_i + i * L) + lane
            tok = gi >> 1
            pos = zero
            new = []
            for e in range(E):
                m = v == e
                mi = jnp.where(m, one, zero)
                cs = plsc.cumsum(mi)
                pos = jnp.where(m, splat(bases[e]) + cs - 1, pos)
                new.append(bases[e] + jnp.sum(mi))
            keep = pos < CAP
            dest = jnp.where(keep, v * CAP + pos, zero)
            val = jnp.where(keep, tok + 1, zero)
            cw = jnp.where(keep, w_v[pl.ds(i * L, L)],
                           jnp.zeros((L,), jnp.float32))
            dest_v[pl.ds(i * L, L)] = dest
            val_v[pl.ds(i * L, L)] = val
            cw_v[pl.ds(i * L, L)] = cw
            return tuple(new)

        lax.fori_loop(0, _CH // L, scan_body, bases)

        pltpu.sync_copy(val_v, inv_sh.at[dest_v], add=True)
        plsc.subcore_barrier()

        pltpu.sync_copy(inv_sh.at[pl.ds(sid * _IVW, _IVW)], inv_v)

        def inv_body(i, _):
            v = inv_v[pl.ds(i * L, L)]
            inv_v[pl.ds(i * L, L)] = jnp.where(v == 0, splat(T), v - 1)
            return 0

        lax.fori_loop(0, _IVW // L, inv_body, 0)
        pltpu.sync_copy(inv_v, inv_hbm.at[pl.ds(sid * _IVW, _IVW)])

        def deint(src_v, dst_v, parity):
            def body(j, _):
                idx = (splat(j * L) + lane) * 2 + parity
                dst_v[pl.ds(j * L, L)] = plsc.load_gather(src_v, [idx])
                return 0
            lax.fori_loop(0, _TW // L, body, 0)

        tb = sid * _TW
        deint(dest_v, dk_v, 0)
        pltpu.sync_copy(dk_v, d0_hbm.at[pl.ds(tb, _TW)])
        deint(dest_v, dk_v, 1)
        pltpu.sync_copy(dk_v, d1_hbm.at[pl.ds(tb, _TW)])
        deint(cw_v, cwk_v, 0)
        pltpu.sync_copy(cwk_v, cw0_hbm.at[pl.ds(tb, _TW)])
        deint(cw_v, cwk_v, 1)
        pltpu.sync_copy(cwk_v, cw1_hbm.at[pl.ds(tb, _TW)])


_RW = ECAP // NW
_RCH = 64


@functools.cache
def _gather_rows_kernel():
  return functools.partial(
    pl.kernel,
    out_type=jax.ShapeDtypeStruct((ECAP, D), jnp.float32),
    mesh=_mesh(),
    compiler_params=pltpu.CompilerParams(needs_layout_passes=False),
    scratch_types=[
        pltpu.VMEM((_RCH,), jnp.int32),
        pltpu.VMEM((_RCH, D), jnp.float32),
        pltpu.SemaphoreType.DMA,
    ],
  )(_gather_rows_body)


def _gather_rows_body(xpad_hbm, inv_hbm, buf_hbm, idx_v, rows_v, sem):
    wid = lax.axis_index("s") * NC + lax.axis_index("c")
    base = wid * _RW
    for c in range(_RW // _RCH):
        pltpu.sync_copy(inv_hbm.at[pl.ds(base + c * _RCH, _RCH)], idx_v)
        pltpu.async_copy(xpad_hbm.at[idx_v], rows_v, sem).wait()
        pltpu.sync_copy(rows_v, buf_hbm.at[pl.ds(base + c * _RCH, _RCH)])


_F = 512


def _ffn_body(buf_ref, wg_ref, wu_ref, wd_ref, y_ref):
    f = pl.program_id(1)
    xb = buf_ref[0].astype(jnp.bfloat16)
    g = jnp.dot(xb, wg_ref[0].astype(jnp.bfloat16),
                preferred_element_type=jnp.float32)
    u = jnp.dot(xb, wu_ref[0].astype(jnp.bfloat16),
                preferred_element_type=jnp.float32)
    h = (g * (1.0 / (1.0 + jnp.exp(-g))) * u).astype(jnp.bfloat16)
    yd = jnp.dot(h, wd_ref[0].astype(jnp.bfloat16),
                 preferred_element_type=jnp.float32)

    @pl.when(f == 0)
    def _():
        y_ref[0] = yd

    @pl.when(f > 0)
    def _():
        y_ref[0] += yd


def _ffn(buf, Wg, Wu, Wd):
    return pl.pallas_call(
        _ffn_body,
        grid=(E, FF // _F),
        in_specs=[
            pl.BlockSpec((1, CAP, D), lambda e, f: (e, 0, 0)),
            pl.BlockSpec((1, D, _F), lambda e, f: (e, 0, f)),
            pl.BlockSpec((1, D, _F), lambda e, f: (e, 0, f)),
            pl.BlockSpec((1, _F, D), lambda e, f: (e, f, 0)),
        ],
        out_specs=pl.BlockSpec((1, CAP, D), lambda e, f: (e, 0, 0)),
        out_shape=jax.ShapeDtypeStruct((E, CAP, D), jnp.float32),
        compiler_params=pltpu.CompilerParams(
            dimension_semantics=("parallel", "arbitrary"),
        ),
    )(buf, Wg, Wu, Wd)


_TKW = T // NW
_TCH = 16


@functools.cache
def _combine_kernel():
  return functools.partial(
    pl.kernel,
    out_type=jax.ShapeDtypeStruct((T, D), jnp.float32),
    mesh=_mesh(),
    compiler_params=pltpu.CompilerParams(needs_layout_passes=False),
    scratch_types=[
        pltpu.VMEM((_TCH,), jnp.int32),
        pltpu.VMEM((_TCH,), jnp.int32),
        pltpu.VMEM((_TCH,), jnp.float32),
        pltpu.VMEM((_TCH,), jnp.float32),
        pltpu.VMEM((_TCH, D), jnp.float32),
        pltpu.VMEM((_TCH, D), jnp.float32),
        pltpu.VMEM((_TCH, D), jnp.float32),
        pltpu.SemaphoreType.DMA,
    ],
  )(_combine_body)


def _combine_body(y_hbm, d0_hbm, d1_hbm, cw0_hbm, cw1_hbm, out_hbm,
             idx0_v, idx1_v, w0_v, w1_v, r0_v, r1_v, out_v, sem):
    wid = lax.axis_index("s") * NC + lax.axis_index("c")
    tb = wid * _TKW
    for c in range(_TKW // _TCH):
        s = tb + c * _TCH
        pltpu.sync_copy(d0_hbm.at[pl.ds(s, _TCH)], idx0_v)
        pltpu.sync_copy(d1_hbm.at[pl.ds(s, _TCH)], idx1_v)
        pltpu.sync_copy(cw0_hbm.at[pl.ds(s, _TCH)], w0_v)
        pltpu.sync_copy(cw1_hbm.at[pl.ds(s, _TCH)], w1_v)
        pltpu.async_copy(y_hbm.at[idx0_v], r0_v, sem).wait()
        pltpu.async_copy(y_hbm.at[idx1_v], r1_v, sem).wait()

        wv0 = w0_v[...]
        wv1 = w1_v[...]
        for j in range(_TCH):
            s0 = wv0[j]
            s1 = wv1[j]

            def vec_body(q, _, j=j, s0=s0, s1=s1):
                sl = pl.ds(q * L, L)
                out_v[j, sl] = r0_v[j, sl] * s0 + r1_v[j, sl] * s1
                return 0

            lax.fori_loop(0, D // L, vec_body, 0)
        pltpu.sync_copy(out_v, out_hbm.at[pl.ds(s, _TCH)])


def kernel(x, Wr, Wg, Wu, Wd):
    wr_padded = jnp.pad(Wr, ((0, 0), (0, 128 - E)))
    eidx, ew = _router(x, wr_padded)
    inv, d0, d1, cw0, cw1 = _dispatch_kernel()(eidx.reshape(-1), ew.reshape(-1))
    xpad = jnp.concatenate([x, jnp.zeros((1, D), x.dtype)], axis=0)
    buf = _gather_rows_kernel()(xpad, inv)
    y = _ffn(buf.reshape(E, CAP, D), Wg, Wu, Wd)
    return _combine_kernel()(y.reshape(ECAP, D), d0, d1, cw0, cw1)


# speedup vs baseline: 1.1685x; 1.0293x over previous
import functools

import jax
import jax.numpy as jnp
from jax import lax
from jax.experimental import pallas as pl
from jax.experimental.pallas import tpu as pltpu
from jax.experimental.pallas import tpu_sc as plsc

T, D, FF, E, K = 4096, 1024, 4096, 8, 2
CAP = int(T * K / E * 1.25)
ECAP = E * CAP
N = T * K

NC, NS, L = 2, 16, 16
NW = NC * NS


@functools.cache
def _mesh():
    return plsc.VectorSubcoreMesh(core_axis_name="c", subcore_axis_name="s",
                                  num_cores=NC, num_subcores=NS)


_BT = 512


def _router_body(x_ref, wr_ref, eidx_ref, ew_ref):
    logits = jnp.dot(x_ref[...].astype(jnp.bfloat16),
                     wr_ref[...].astype(jnp.bfloat16),
                     preferred_element_type=jnp.float32)
    col = lax.broadcasted_iota(jnp.int32, (_BT, 128), 1)
    neg = jnp.float32(-jnp.inf)
    lg = jnp.where(col < E, logits, neg)
    m1 = jnp.max(lg, axis=1, keepdims=True)
    i1 = jnp.min(jnp.where(lg == m1, col, 128), axis=1, keepdims=True)
    lg2 = jnp.where(col == i1, neg, lg)
    m2 = jnp.max(lg2, axis=1, keepdims=True)
    i2 = jnp.min(jnp.where(lg2 == m2, col, 128), axis=1, keepdims=True)
    w1 = 1.0 / (1.0 + jnp.exp(m2 - m1))
    eidx_ref[...] = jnp.concatenate([i1, i2], axis=1)
    ew_ref[...] = jnp.concatenate([w1, 1.0 - w1], axis=1)


def _router(x, wr_padded):
    return pl.pallas_call(
        _router_body,
        grid=(T // _BT,),
        in_specs=[
            pl.BlockSpec((_BT, D), lambda i: (i, 0)),
            pl.BlockSpec((D, 128), lambda i: (0, 0)),
        ],
        out_specs=[
            pl.BlockSpec((_BT, 2), lambda i: (i, 0)),
            pl.BlockSpec((_BT, 2), lambda i: (i, 0)),
        ],
        out_shape=[
            jax.ShapeDtypeStruct((T, 2), jnp.int32),
            jax.ShapeDtypeStruct((T, 2), jnp.float32),
        ],
    )(x, wr_padded)


_CH = N // NS
_IVW = ECAP // NS
_TW = T // NS


@functools.cache
def _dispatch_kernel():
  return functools.partial(
    pl.kernel,
    out_type=[
        jax.ShapeDtypeStruct((ECAP,), jnp.int32),
        jax.ShapeDtypeStruct((ECAP,), jnp.float32),
        jax.ShapeDtypeStruct((T,), jnp.int32),
        jax.ShapeDtypeStruct((T,), jnp.int32),
    ],
    mesh=_mesh(),
    compiler_params=pltpu.CompilerParams(needs_layout_passes=False),
    scratch_types=[
        pltpu.VMEM((_CH,), jnp.int32),
        pltpu.VMEM((_CH,), jnp.float32),
        pltpu.VMEM((L,), jnp.int32),
        pltpu.VMEM((NS * L,), jnp.int32),
        pltpu.VMEM((_CH,), jnp.int32),
        pltpu.VMEM((_CH,), jnp.int32),
        pltpu.VMEM((_CH,), jnp.float32),
        pltpu.VMEM((_IVW,), jnp.int32),
        pltpu.VMEM((_IVW,), jnp.float32),
        pltpu.VMEM((_TW,), jnp.int32),
        pltpu.VMEM_SHARED((NS * L,), jnp.int32),
        pltpu.VMEM_SHARED((ECAP,), jnp.int32),
        pltpu.VMEM_SHARED((ECAP,), jnp.float32),
    ],
  )(_dispatch_body)


def _dispatch_body(e_hbm, w_hbm, inv_hbm, wsl_hbm, d0_hbm, d1_hbm,
                   ids_v, w_v, mycnt_v, allcnt_v, dest_v, val_v, cw_v, inv_v,
                   wz_v, dk_v, counts_sh, inv_sh, wsl_sh):
    cid = lax.axis_index("c")
    sid = lax.axis_index("s")
    lane = lax.iota(jnp.int32, L)
    one = jnp.full((L,), 1, jnp.int32)
    zero = jnp.zeros((L,), jnp.int32)
    fzero = jnp.zeros((L,), jnp.float32)

    def splat(s):
        return jnp.broadcast_to(s, (L,))

    @pl.when(cid == 0)
    def _():
        base_i = sid * _CH
        pltpu.sync_copy(e_hbm.at[pl.ds(base_i, _CH)], ids_v)
        pltpu.sync_copy(w_hbm.at[pl.ds(base_i, _CH)], w_v)

        def cnt_body(i, accs):
            v = ids_v[pl.ds(i * L, L)]
            return tuple(accs[e] + jnp.where(v == e, one, zero)
                         for e in range(E))

        accs = lax.fori_loop(0, _CH // L, cnt_body,
                             tuple(jnp.zeros((L,), jnp.int32) for _ in range(E)))
        cnt = zero
        for e in range(E):
            cnt = jnp.where(lane == e, splat(jnp.sum(accs[e])), cnt)
        mycnt_v[...] = cnt
        pltpu.sync_copy(mycnt_v, counts_sh.at[pl.ds(sid * L, L)])

        def zero_body(i, _):
            inv_v[pl.ds(i * L, L)] = zero
            wz_v[pl.ds(i * L, L)] = fzero
            return 0

        lax.fori_loop(0, _IVW // L, zero_body, 0)
        pltpu.sync_copy(inv_v, inv_sh.at[pl.ds(sid * _IVW, _IVW)])
        pltpu.sync_copy(wz_v, wsl_sh.at[pl.ds(sid * _IVW, _IVW)])
        plsc.subcore_barrier()

        pltpu.sync_copy(counts_sh, allcnt_v)
        offv = zero
        totv = zero
        for w in range(NS):
            row = allcnt_v[pl.ds(w * L, L)]
            offv = offv + jnp.where(jnp.broadcast_to(sid > w, (L,)), row, zero)
            totv = totv + row
        bases = tuple(jnp.sum(jnp.where(lane == e, offv, zero))
                      for e in range(E))
        cand = jnp.where((lane < E) & (totv < CAP), lane * CAP + totv,
                         splat(2 * ECAP))
        empty_slot = jnp.min(cand)

        def scan_body(i, bases):
            v = ids_v[pl.ds(i * L, L)]
            gi = splat(base_i + i * L) + lane
            tok = gi >> 1
            pos = zero
            new = []
            for e in range(E):
                m = v == e
                mi = jnp.where(m, one, zero)
                cs = plsc.cumsum(mi)
                pos = jnp.where(m, splat(bases[e]) + cs - 1, pos)
                new.append(bases[e] + jnp.sum(mi))
            keep = pos < CAP
            dest = jnp.where(keep, v * CAP + pos, splat(empty_slot))
            val = jnp.where(keep, tok + 1, zero)
            cw = jnp.where(keep, w_v[pl.ds(i * L, L)], fzero)
            dest_v[pl.ds(i * L, L)] = dest
            val_v[pl.ds(i * L, L)] = val
            cw_v[pl.ds(i * L, L)] = cw
            return tuple(new)

        lax.fori_loop(0, _CH // L, scan_body, bases)

        pltpu.sync_copy(val_v, inv_sh.at[dest_v], add=True)
        pltpu.sync_copy(cw_v, wsl_sh.at[dest_v], add=True)
        plsc.subcore_barrier()

        pltpu.sync_copy(inv_sh.at[pl.ds(sid * _IVW, _IVW)], inv_v)
        pltpu.sync_copy(wsl_sh.at[pl.ds(sid * _IVW, _IVW)], wz_v)

        def inv_body(i, _):
            v = inv_v[pl.ds(i * L, L)]
            inv_v[pl.ds(i * L, L)] = jnp.where(v == 0, splat(T), v - 1)
            return 0

        lax.fori_loop(0, _IVW // L, inv_body, 0)
        pltpu.sync_copy(inv_v, inv_hbm.at[pl.ds(sid * _IVW, _IVW)])
        pltpu.sync_copy(wz_v, wsl_hbm.at[pl.ds(sid * _IVW, _IVW)])

        def deint(parity):
            def body(j, _):
                idx = (splat(j * L) + lane) * 2 + parity
                dk_v[pl.ds(j * L, L)] = plsc.load_gather(dest_v, [idx])
                return 0
            lax.fori_loop(0, _TW // L, body, 0)

        tb = sid * _TW
        deint(0)
        pltpu.sync_copy(dk_v, d0_hbm.at[pl.ds(tb, _TW)])
        deint(1)
        pltpu.sync_copy(dk_v, d1_hbm.at[pl.ds(tb, _TW)])


_RW = ECAP // NW
_RCH = 40
_RNC = _RW // _RCH


@functools.cache
def _gather_rows_kernel():
  return functools.partial(
    pl.kernel,
    out_type=jax.ShapeDtypeStruct((ECAP, D), jnp.float32),
    mesh=_mesh(),
    compiler_params=pltpu.CompilerParams(needs_layout_passes=False),
    scratch_types=[
        pltpu.VMEM((_RW,), jnp.int32),
        pltpu.VMEM((_RCH, D), jnp.float32),
        pltpu.VMEM((_RCH, D), jnp.float32),
        pltpu.SemaphoreType.DMA,
        pltpu.SemaphoreType.DMA,
        pltpu.SemaphoreType.DMA,
        pltpu.SemaphoreType.DMA,
    ],
  )(_gather_rows_body)


def _gather_rows_body(xpad_hbm, inv_hbm, buf_hbm, idx_v, rows0, rows1,
                      gs0, gs1, os0, os1):
    wid = lax.axis_index("s") * NC + lax.axis_index("c")
    base = wid * _RW
    rows = (rows0, rows1)
    gsem = (gs0, gs1)
    osem = (os0, os1)
    pltpu.sync_copy(inv_hbm.at[pl.ds(base, _RW)], idx_v)

    def gstart(c):
        b = c & 1
        return pltpu.async_copy(
            xpad_hbm.at[idx_v.at[pl.ds(c * _RCH, _RCH)]], rows[b], gsem[b])

    g = {0: gstart(0), 1: gstart(1)}
    o = {}
    owaited = set()
    for c in range(_RNC):
        b = c & 1
        g[c].wait()
        o[c] = pltpu.async_copy(
            rows[b], buf_hbm.at[pl.ds(base + c * _RCH, _RCH)], osem[b])
        if c + 2 < _RNC:
            o[c].wait()
            owaited.add(c)
            g[c + 2] = gstart(c + 2)
    for c in range(_RNC):
        if c not in owaited:
            o[c].wait()


_F = 512
_NF = FF // _F


def _ffn_body(buf_ref, wsl_ref, wg_ref, wu_ref, wd_ref, y_ref):
    f = pl.program_id(1)
    xb = buf_ref[0].astype(jnp.bfloat16)
    g = jnp.dot(xb, wg_ref[0].astype(jnp.bfloat16),
                preferred_element_type=jnp.float32)
    u = jnp.dot(xb, wu_ref[0].astype(jnp.bfloat16),
                preferred_element_type=jnp.float32)
    h = (g * (1.0 / (1.0 + jnp.exp(-g))) * u).astype(jnp.bfloat16)
    yd = jnp.dot(h, wd_ref[0].astype(jnp.bfloat16),
                 preferred_element_type=jnp.float32)

    @pl.when(f == 0)
    def _():
        y_ref[0] = yd

    @pl.when((f > 0) & (f < _NF - 1))
    def _():
        y_ref[0] += yd

    @pl.when(f == _NF - 1)
    def _():
        y_ref[0] = (y_ref[0] + yd) * wsl_ref[0]


def _ffn(buf, wsl, Wg, Wu, Wd):
    return pl.pallas_call(
        _ffn_body,
        grid=(E, _NF),
        in_specs=[
            pl.BlockSpec((1, CAP, D), lambda e, f: (e, 0, 0)),
            pl.BlockSpec((1, CAP, 1), lambda e, f: (e, 0, 0)),
            pl.BlockSpec((1, D, _F), lambda e, f: (e, 0, f)),
            pl.BlockSpec((1, D, _F), lambda e, f: (e, 0, f)),
            pl.BlockSpec((1, _F, D), lambda e, f: (e, f, 0)),
        ],
        out_specs=pl.BlockSpec((1, CAP, D), lambda e, f: (e, 0, 0)),
        out_shape=jax.ShapeDtypeStruct((E, CAP, D), jnp.float32),
        compiler_params=pltpu.CompilerParams(
            dimension_semantics=("parallel", "arbitrary"),
        ),
    )(buf, wsl, Wg, Wu, Wd)


_TKW = T // NW
_TCH = 16
_TNC = _TKW // _TCH


@functools.cache
def _combine_kernel():
  return functools.partial(
    pl.kernel,
    out_type=jax.ShapeDtypeStruct((T, D), jnp.float32),
    mesh=_mesh(),
    compiler_params=pltpu.CompilerParams(needs_layout_passes=False),
    scratch_types=[
        pltpu.VMEM((_TKW,), jnp.int32),
        pltpu.VMEM((_TKW,), jnp.int32),
        pltpu.VMEM((_TCH, D), jnp.float32),
        pltpu.VMEM((_TCH, D), jnp.float32),
        pltpu.VMEM((_TCH, D), jnp.float32),
        pltpu.VMEM((_TCH, D), jnp.float32),
        pltpu.SemaphoreType.DMA,
        pltpu.SemaphoreType.DMA,
        pltpu.SemaphoreType.DMA,
        pltpu.SemaphoreType.DMA,
        pltpu.SemaphoreType.DMA,
        pltpu.SemaphoreType.DMA,
    ],
  )(_combine_body)


def _combine_body(y_hbm, d0_hbm, d1_hbm, out_hbm,
                  idx0_v, idx1_v, ra0, rb0, ra1, rb1,
                  gsa0, gsb0, gsa1, gsb1, os0, os1):
    wid = lax.axis_index("s") * NC + lax.axis_index("c")
    tb = wid * _TKW
    ra = (ra0, ra1)
    rb = (rb0, rb1)
    gsa = (gsa0, gsa1)
    gsb = (gsb0, gsb1)
    osem = (os0, os1)
    pltpu.sync_copy(d0_hbm.at[pl.ds(tb, _TKW)], idx0_v)
    pltpu.sync_copy(d1_hbm.at[pl.ds(tb, _TKW)], idx1_v)

    def gstart(c):
        b = c & 1
        a1 = pltpu.async_copy(
            y_hbm.at[idx0_v.at[pl.ds(c * _TCH, _TCH)]], ra[b], gsa[b])
        a2 = pltpu.async_copy(
            y_hbm.at[idx1_v.at[pl.ds(c * _TCH, _TCH)]], rb[b], gsb[b])
        return (a1, a2)

    g = {0: gstart(0), 1: gstart(1)}
    o = {}
    owaited = set()
    for c in range(_TNC):
        b = c & 1
        g[c][0].wait()
        g[c][1].wait()

        for j in range(_TCH):
            def vec_body(q, _, j=j):
                sl = pl.ds(q * L, L)
                ra[b][j, sl] = ra[b][j, sl] + rb[b][j, sl]
                return 0
            lax.fori_loop(0, D // L, vec_body, 0)

        o[c] = pltpu.async_copy(
            ra[b], out_hbm.at[pl.ds(tb + c * _TCH, _TCH)], osem[b])
        if c + 2 < _TNC:
            o[c].wait()
            owaited.add(c)
            g[c + 2] = gstart(c + 2)
    for c in range(_TNC):
        if c not in owaited:
            o[c].wait()


def kernel(x, Wr, Wg, Wu, Wd):
    wr_padded = jnp.pad(Wr, ((0, 0), (0, 128 - E)))
    eidx, ew = _router(x, wr_padded)
    inv, wsl, d0, d1 = _dispatch_kernel()(eidx.reshape(-1), ew.reshape(-1))
    xpad = jnp.concatenate([x, jnp.zeros((1, D), x.dtype)], axis=0)
    buf = _gather_rows_kernel()(xpad, inv)
    y = _ffn(buf.reshape(E, CAP, D), wsl.reshape(E, CAP, 1), Wg, Wu, Wd)
    return _combine_kernel()(y.reshape(ECAP, D), d0, d1)
